# baseline (speedup 1.0000x reference)
import functools

import jax
import jax.numpy as jnp
from jax import lax
from jax.experimental import pallas as pl
from jax.experimental.pallas import tpu as pltpu
from jax.experimental.pallas import tpu_sc as plsc

VOCAB = 1000
NC, NS = 2, 16
NW = NC * NS
N_TOK = 1024 * 50
PER_W = N_TOK // NW
K = 32
N_CHUNK = PER_W // K
N_PAIR = N_CHUNK // 2


def _gather_body(idx_hbm, table_hbm, out_hbm, idx_v, buf0, buf1, sem0, sem1):
    wid = lax.axis_index("s") * NC + lax.axis_index("c")
    base = wid * PER_W
    pltpu.sync_copy(idx_hbm.at[pl.ds(base, PER_W)], idx_v)

    def start(cc, buf, sem):
        pltpu.async_copy(table_hbm.at[idx_v.at[pl.ds(cc * K, K)]], buf, sem)

    def wait(buf, sem):
        pltpu.make_async_copy(table_hbm.at[idx_v.at[pl.ds(0, K)]], buf, sem).wait()

    start(0, buf0, sem0)

    def pair(g, carry):
        cc = 2 * g
        start(cc + 1, buf1, sem1)
        wait(buf0, sem0)
        pltpu.sync_copy(buf0, out_hbm.at[pl.ds(base + cc * K, K), :])

        @pl.when(g < N_PAIR - 1)
        def _():
            start(cc + 2, buf0, sem0)

        wait(buf1, sem1)
        pltpu.sync_copy(buf1, out_hbm.at[pl.ds(base + (cc + 1) * K, K), :])
        return carry

    lax.fori_loop(0, N_PAIR, pair, 0)


_sc_gather = functools.partial(
    pl.kernel,
    out_type=jax.ShapeDtypeStruct((N_TOK, VOCAB), jnp.float32),
    mesh=plsc.VectorSubcoreMesh(
        core_axis_name="c", subcore_axis_name="s", num_cores=NC, num_subcores=NS
    ),
    scratch_types=[
        pltpu.VMEM((PER_W,), jnp.int32),
        pltpu.VMEM((K, VOCAB), jnp.float32),
        pltpu.VMEM((K, VOCAB), jnp.float32),
        pltpu.SemaphoreType.DMA,
        pltpu.SemaphoreType.DMA,
    ],
    compiler_params=pltpu.CompilerParams(use_tc_tiling_on_sc=False),
)(_gather_body)


RB = 512
N_STEP = N_TOK // RB


def _loss_body(lg_ref, tg_ref, loss_ref, acc_ref):
    i = pl.program_id(0)

    @pl.when(i == 0)
    def _():
        acc_ref[0, 0] = 0.0

    l = lg_ref[...]
    m = jnp.max(l, axis=1, keepdims=True)
    s = jnp.sum(jnp.exp(l - m), axis=1, keepdims=True)
    lse = m + jnp.log(s)
    lane = lax.broadcasted_iota(jnp.int32, l.shape, 1)
    tv = jnp.sum(jnp.where(lane == tg_ref[...], l, 0.0), axis=1, keepdims=True)
    acc_ref[0, 0] += jnp.sum(lse - tv)

    @pl.when(i == N_STEP - 1)
    def _():
        loss_ref[0, 0] = acc_ref[0, 0] / N_TOK


def _tc_loss(logits2d, tgt2d):
    return pl.pallas_call(
        _loss_body,
        grid=(N_STEP,),
        in_specs=[
            pl.BlockSpec((RB, VOCAB), lambda i: (i, 0)),
            pl.BlockSpec((RB, 1), lambda i: (i, 0)),
        ],
        out_specs=pl.BlockSpec(memory_space=pltpu.SMEM),
        out_shape=jax.ShapeDtypeStruct((1, 1), jnp.float32),
        scratch_shapes=[pltpu.SMEM((1, 1), jnp.float32)],
    )(logits2d, tgt2d)


def kernel(idx, target, table):
    B, T = idx.shape
    idx_flat = idx.reshape(N_TOK).astype(jnp.int32)
    logits2d = _sc_gather(idx_flat, table)
    loss = _tc_loss(logits2d, target.reshape(N_TOK, 1))
    return logits2d.reshape(B, T, VOCAB), loss[0, 0]


# speedup vs baseline: 1.4427x; 1.4427x over previous
import functools

import jax
import jax.numpy as jnp
from jax import lax
from jax.experimental import pallas as pl
from jax.experimental.pallas import tpu as pltpu
from jax.experimental.pallas import tpu_sc as plsc

VOCAB = 1000
NC, NS = 2, 16
NW = NC * NS
N_TOK = 1024 * 50
PER_W = N_TOK // NW
K = 32
N_CHUNK = PER_W // K
N_PAIR = N_CHUNK // 2
LANES = 16


def _lse_body(tbl_ref, lse_ref):
    l = tbl_ref[...]
    m = jnp.max(l, axis=1, keepdims=True)
    lse_ref[...] = m + jnp.log(jnp.sum(jnp.exp(l - m), axis=1, keepdims=True))


def _tc_lse(table):
    return pl.pallas_call(
        _lse_body,
        out_shape=jax.ShapeDtypeStruct((VOCAB, 1), jnp.float32),
    )(table)


def _gather_body(idx_hbm, tgt_hbm, lse_hbm, table_hbm, out_hbm, part_hbm,
                 idx_v, tgt_v, lse_v, part_v, buf0, buf1, sem0, sem1):
    wid = lax.axis_index("s") * NC + lax.axis_index("c")
    base = wid * PER_W
    pltpu.sync_copy(idx_hbm.at[pl.ds(base, PER_W)], idx_v)
    pltpu.sync_copy(tgt_hbm.at[pl.ds(base, PER_W)], tgt_v)
    pltpu.sync_copy(lse_hbm, lse_v)

    def start(cc, buf, sem):
        pltpu.async_copy(table_hbm.at[idx_v.at[pl.ds(cc * K, K)]], buf, sem)

    def wait(buf, sem):
        pltpu.make_async_copy(table_hbm.at[idx_v.at[pl.ds(0, K)]], buf, sem).wait()

    def chunk_loss(cc, buf, acc):
        for q in range(K // LANES):
            off = cc * K + q * LANES
            rid = lax.iota(jnp.int32, LANES) + q * LANES
            tg16 = tgt_v[pl.ds(off, LANES)]
            ix16 = idx_v[pl.ds(off, LANES)]
            tv = plsc.load_gather(buf, [rid, tg16])
            lsev = plsc.load_gather(lse_v, [ix16])
            acc = acc + (lsev - tv)
        return acc

    start(0, buf0, sem0)

    def pair(g, acc):
        cc = 2 * g
        start(cc + 1, buf1, sem1)
        wait(buf0, sem0)
        acc = chunk_loss(cc, buf0, acc)
        pltpu.sync_copy(buf0, out_hbm.at[pl.ds(base + cc * K, K), :])

        @pl.when(g < N_PAIR - 1)
        def _():
            start(cc + 2, buf0, sem0)

        wait(buf1, sem1)
        acc = chunk_loss(cc + 1, buf1, acc)
        pltpu.sync_copy(buf1, out_hbm.at[pl.ds(base + (cc + 1) * K, K), :])
        return acc

    acc = lax.fori_loop(0, N_PAIR, pair, jnp.zeros((LANES,), jnp.float32))
    part_v[...] = acc
    pltpu.sync_copy(part_v, part_hbm.at[pl.ds(wid * LANES, LANES)])


_sc_gather = functools.partial(
    pl.kernel,
    out_type=(
        jax.ShapeDtypeStruct((N_TOK, VOCAB), jnp.float32),
        jax.ShapeDtypeStruct((NW * LANES,), jnp.float32),
    ),
    mesh=plsc.VectorSubcoreMesh(
        core_axis_name="c", subcore_axis_name="s", num_cores=NC, num_subcores=NS
    ),
    scratch_types=[
        pltpu.VMEM((PER_W,), jnp.int32),
        pltpu.VMEM((PER_W,), jnp.int32),
        pltpu.VMEM((VOCAB,), jnp.float32),
        pltpu.VMEM((LANES,), jnp.float32),
        pltpu.VMEM((K, VOCAB), jnp.float32),
        pltpu.VMEM((K, VOCAB), jnp.float32),
        pltpu.SemaphoreType.DMA,
        pltpu.SemaphoreType.DMA,
    ],
    compiler_params=pltpu.CompilerParams(
        use_tc_tiling_on_sc=False, needs_layout_passes=False
    ),
)(_gather_body)


def _final_body(part_ref, loss_ref):
    loss_ref[0, 0] = jnp.sum(part_ref[...]) / N_TOK


def _tc_final(partials):
    return pl.pallas_call(
        _final_body,
        out_specs=pl.BlockSpec(memory_space=pltpu.SMEM),
        out_shape=jax.ShapeDtypeStruct((1, 1), jnp.float32),
    )(partials)


def kernel(idx, target, table):
    B, T = idx.shape
    idx_flat = idx.reshape(N_TOK).astype(jnp.int32)
    tgt_flat = target.reshape(N_TOK).astype(jnp.int32)
    lse = _tc_lse(table).reshape(VOCAB)
    logits2d, partials = _sc_gather(idx_flat, tgt_flat, lse, table)
    loss = _tc_final(partials.reshape(NW, LANES))
    return logits2d.reshape(B, T, VOCAB), loss[0, 0]
